# baseline (speedup 1.0000x reference)
import functools

import jax
import jax.numpy as jnp
from jax import lax
from jax.experimental import pallas as pl
from jax.experimental.pallas import tpu as pltpu
from jax.experimental.pallas import tpu_sc as plsc

N_CLS = 1000
N_CTX = 16
DIM = 512
SEQ = 77
SUF = SEQ - 1 - N_CTX


def kernel(ctx, token_prefix, token_suffix):
    info = plsc.get_sparse_core_info()
    nc, ns = info.num_cores, info.num_subcores
    nw = nc * ns
    cpw = (N_CLS + nw - 1) // nw
    blk = SEQ * DIM

    mesh = plsc.VectorSubcoreMesh(core_axis_name="c", subcore_axis_name="s")

    @functools.partial(
        pl.kernel,
        mesh=mesh,
        out_type=jax.ShapeDtypeStruct((N_CLS, blk), jnp.float32),
        scratch_types=[
            pltpu.VMEM((blk,), jnp.float32),
        ],
    )
    def prompt_assemble(ctx_hbm, pre_hbm, suf_hbm, out_hbm, buf):
        cid = lax.axis_index("c")
        sid = lax.axis_index("s")
        wid = sid * nc + cid

        pltpu.sync_copy(ctx_hbm, buf.at[pl.ds(DIM, N_CTX * DIM)])

        for i in range(cpw):
            c = i * nw + wid

            @pl.when(c < N_CLS)
            def _():
                pltpu.sync_copy(pre_hbm.at[c], buf.at[pl.ds(0, DIM)])
                pltpu.sync_copy(suf_hbm.at[c],
                                buf.at[pl.ds((1 + N_CTX) * DIM, SUF * DIM)])
                pltpu.sync_copy(buf, out_hbm.at[c])

    out = prompt_assemble(
        ctx.reshape(N_CTX * DIM),
        token_prefix.reshape(N_CLS, DIM),
        token_suffix.reshape(N_CLS, SUF * DIM),
    )
    return out.reshape(N_CLS, SEQ, DIM)


# speedup vs baseline: 1.0125x; 1.0125x over previous
import functools

import jax
import jax.numpy as jnp
from jax import lax
from jax.experimental import pallas as pl
from jax.experimental.pallas import tpu as pltpu
from jax.experimental.pallas import tpu_sc as plsc

N_CLS = 1000
N_CTX = 16
DIM = 512
SEQ = 77
SUF = SEQ - 1 - N_CTX


def kernel(ctx, token_prefix, token_suffix):
    info = plsc.get_sparse_core_info()
    nc, ns = info.num_cores, info.num_subcores
    nw = nc * ns
    cpw = (N_CLS + nw - 1) // nw
    blk = SEQ * DIM

    mesh = plsc.VectorSubcoreMesh(core_axis_name="c", subcore_axis_name="s")

    n_full = N_CLS // nw
    n_rem = N_CLS - n_full * nw

    @functools.partial(
        pl.kernel,
        mesh=mesh,
        out_type=jax.ShapeDtypeStruct((N_CLS, blk), jnp.float32),
        scratch_types=[
            pltpu.VMEM((2, blk), jnp.float32),
            pltpu.SemaphoreType.DMA,
            pltpu.SemaphoreType.DMA,
        ],
    )
    def prompt_assemble(ctx_hbm, pre_hbm, suf_hbm, out_hbm, buf, sem_in,
                        sem_out):
        cid = lax.axis_index("c")
        sid = lax.axis_index("s")
        wid = sid * nc + cid

        pltpu.sync_copy(ctx_hbm, buf.at[0, pl.ds(DIM, N_CTX * DIM)])
        pltpu.sync_copy(ctx_hbm, buf.at[1, pl.ds(DIM, N_CTX * DIM)])

        def gather(i):
            s = i % 2
            c = i * nw + wid
            return (
                pltpu.make_async_copy(pre_hbm.at[c], buf.at[s, pl.ds(0, DIM)],
                                      sem_in),
                pltpu.make_async_copy(
                    suf_hbm.at[c],
                    buf.at[s, pl.ds((1 + N_CTX) * DIM, SUF * DIM)], sem_in),
            )

        def scatter(i):
            c = i * nw + wid
            return pltpu.make_async_copy(buf.at[i % 2], out_hbm.at[c],
                                         sem_out)

        def start(hs):
            for h in (hs if isinstance(hs, tuple) else (hs,)):
                h.start()

        def wait(hs):
            for h in (hs if isinstance(hs, tuple) else (hs,)):
                h.wait()

        start(gather(0))
        for i in range(n_full):
            wait(gather(i))
            start(scatter(i))
            if i + 1 < n_full:
                if i >= 1:
                    wait(scatter(i - 1))
                start(gather(i + 1))
        if n_full >= 2:
            wait(scatter(n_full - 2))
        wait(scatter(n_full - 1))

        @pl.when(wid < n_rem)
        def _():
            c = n_full * nw + wid
            pltpu.sync_copy(pre_hbm.at[c], buf.at[0, pl.ds(0, DIM)])
            pltpu.sync_copy(suf_hbm.at[c],
                            buf.at[0, pl.ds((1 + N_CTX) * DIM, SUF * DIM)])
            pltpu.sync_copy(buf.at[0], out_hbm.at[c])

    out = prompt_assemble(
        ctx.reshape(N_CTX * DIM),
        token_prefix.reshape(N_CLS, DIM),
        token_suffix.reshape(N_CLS, SUF * DIM),
    )
    return out.reshape(N_CLS, SEQ, DIM)


# speedup vs baseline: 1.1581x; 1.1438x over previous
import functools

import jax
import jax.numpy as jnp
from jax import lax
from jax.experimental import pallas as pl
from jax.experimental.pallas import tpu as pltpu
from jax.experimental.pallas import tpu_sc as plsc

N_CLS = 1000
N_CTX = 16
DIM = 512
SEQ = 77
SUF = SEQ - 1 - N_CTX


def kernel(ctx, token_prefix, token_suffix):
    info = plsc.get_sparse_core_info()
    nc, ns = info.num_cores, info.num_subcores
    nw = nc * ns
    n_long = N_CLS % nw

    mesh = plsc.VectorSubcoreMesh(core_axis_name="c", subcore_axis_name="s")

    @functools.partial(
        pl.kernel,
        mesh=mesh,
        out_type=jax.ShapeDtypeStruct((N_CLS, SEQ, DIM), jnp.float32),
        scratch_types=[
            pltpu.VMEM((2, SEQ, DIM), jnp.float32),
            pltpu.VMEM((SUF, DIM), jnp.float32),
            pltpu.VMEM((N_CTX, DIM), jnp.float32),
            pltpu.SemaphoreType.DMA,
            pltpu.SemaphoreType.DMA,
            pltpu.SemaphoreType.DMA,
        ],
    )
    def prompt_assemble(ctx_hbm, pre_hbm, suf_hbm, out_hbm, out_buf, suf_buf,
                        ctx_buf, sem_suf, sem_pre, sem_out):
        cid = lax.axis_index("c")
        sid = lax.axis_index("s")
        wid = sid * nc + cid
        n = jnp.where(wid < n_long, N_CLS // nw + 1, N_CLS // nw)

        def suf_copy(i):
            return pltpu.make_async_copy(suf_hbm.at[i * nw + wid], suf_buf,
                                         sem_suf)

        def pre_copy(i, s):
            return pltpu.make_async_copy(pre_hbm.at[i * nw + wid],
                                         out_buf.at[s, pl.ds(0, 1)], sem_pre)

        def out_copy(i, s):
            return pltpu.make_async_copy(out_buf.at[s],
                                         out_hbm.at[i * nw + wid], sem_out)

        pltpu.sync_copy(ctx_hbm, ctx_buf)
        for s in (0, 1):

            @pl.loop(0, N_CTX)
            def _(r):
                for l in range(0, DIM, 16):
                    out_buf[s, 1 + r, pl.ds(l, 16)] = ctx_buf[r, pl.ds(l, 16)]

        suf_copy(0).start()

        @pl.loop(0, n)
        def _(i):
            s = lax.rem(i, 2)
            suf_copy(i).wait()

            @pl.when(i >= 2)
            def _():
                out_copy(i - 2, s).wait()

            pre_copy(i, s).start()

            @pl.loop(0, SUF)
            def _(r):
                for l in range(0, DIM, 16):
                    out_buf[s, 17 + r, pl.ds(l, 16)] = suf_buf[r,
                                                               pl.ds(l, 16)]

            @pl.when(i + 1 < n)
            def _():
                suf_copy(i + 1).start()

            pre_copy(i, s).wait()
            out_copy(i, s).start()

        out_copy(n - 2, lax.rem(n - 2, 2)).wait()
        out_copy(n - 1, lax.rem(n - 1, 2)).wait()

    return prompt_assemble(ctx, token_prefix, token_suffix)


# speedup vs baseline: 4.0968x; 3.5374x over previous
import functools

import jax
import jax.numpy as jnp
from jax import lax
from jax.experimental import pallas as pl
from jax.experimental.pallas import tpu as pltpu
from jax.experimental.pallas import tpu_sc as plsc

N_CLS = 1000
N_CTX = 16
DIM = 512
SEQ = 77
SUF = SEQ - 1 - N_CTX
RB = 32


def kernel(ctx, token_prefix, token_suffix):
    info = plsc.get_sparse_core_info()
    nc, ns = info.num_cores, info.num_subcores
    nw = nc * ns
    tail_rows = N_CLS - (nw - 1) * RB

    mesh = plsc.VectorSubcoreMesh(core_axis_name="c", subcore_axis_name="s")

    @functools.partial(
        pl.kernel,
        mesh=mesh,
        out_type=jax.ShapeDtypeStruct((SEQ, N_CLS, DIM), jnp.float32),
        scratch_types=[
            pltpu.VMEM((N_CTX, DIM), jnp.float32),
            pltpu.VMEM((N_CTX, 8, DIM), jnp.float32),
            pltpu.VMEM((2, RB, DIM), jnp.float32),
            pltpu.VMEM((RB, DIM), jnp.float32),
            pltpu.SemaphoreType.DMA,
            pltpu.SemaphoreType.DMA,
            pltpu.SemaphoreType.DMA,
            pltpu.SemaphoreType.DMA,
        ],
    )
    def prompt_assemble(ctx_hbm, pre_hbm, suf_hbm, out_hbm, ctx_buf, brd_buf,
                        sbuf, pbuf, sem_in, sem_out, sem_pre, sem_ctx):
        cid = lax.axis_index("c")
        sid = lax.axis_index("s")
        wid = sid * nc + cid
        r0 = wid * RB

        pltpu.sync_copy(ctx_hbm, ctx_buf)

        @pl.loop(0, N_CTX)
        def _(j):
            for l in range(0, DIM, 16):
                v = ctx_buf[j, pl.ds(l, 16)]
                for k in range(8):
                    brd_buf[j, k, pl.ds(l, 16)] = v

        def do_rows(nr):
            ctx_copies = [
                pltpu.make_async_copy(
                    brd_buf.at[j],
                    out_hbm.at[1 + j, pl.ds(r0 + 8 * k, 8)], sem_ctx)
                for j in range(N_CTX) for k in range(nr // 8)
            ]
            for h in ctx_copies:
                h.start()

            pg = pltpu.make_async_copy(pre_hbm.at[pl.ds(r0, nr)],
                                       pbuf.at[pl.ds(0, nr)], sem_pre)
            pg.start()
            pg.wait()
            ps = pltpu.make_async_copy(pbuf.at[pl.ds(0, nr)],
                                       out_hbm.at[0, pl.ds(r0, nr)], sem_pre)
            ps.start()

            def s_gather(s, slot):
                return pltpu.make_async_copy(suf_hbm.at[s, pl.ds(r0, nr)],
                                             sbuf.at[slot, pl.ds(0, nr)],
                                             sem_in)

            def s_scatter(s, slot):
                return pltpu.make_async_copy(sbuf.at[slot, pl.ds(0, nr)],
                                             out_hbm.at[17 + s,
                                                        pl.ds(r0, nr)],
                                             sem_out)

            s_gather(0, 0).start()

            @pl.loop(0, SUF)
            def _(s):
                slot = lax.rem(s, 2)
                s_gather(s, slot).wait()

                @pl.when(s >= 1)
                def _():
                    s_scatter(s - 1, 1 - slot).wait()

                @pl.when(s + 1 < SUF)
                def _():
                    s_gather(s + 1, 1 - slot).start()

                s_scatter(s, slot).start()

            s_scatter(SUF - 1, lax.rem(SUF - 1, 2)).wait()
            ps.wait()
            for h in ctx_copies:
                h.wait()

        @pl.when(wid < nw - 1)
        def _():
            do_rows(RB)

        @pl.when(wid == nw - 1)
        def _():
            do_rows(tail_rows)

    pre2 = token_prefix.reshape(N_CLS, DIM)
    suf_t = jnp.transpose(token_suffix, (1, 0, 2))
    out_t = prompt_assemble(ctx, pre2, suf_t)
    return jnp.transpose(out_t, (1, 0, 2))
